# Initial kernel scaffold; baseline (speedup 1.0000x reference)
import jax
import jax.numpy as jnp
from jax.experimental import pallas as pl


def kernel(precursor_mz, charge, charge_table, W1, b1, W2, b2):
    raise NotImplementedError("write your pallas kernel here")


# speedup vs baseline: 2.9868x; 2.9868x over previous
import functools

import jax
import jax.numpy as jnp
from jax.experimental import pallas as pl
from jax.experimental.pallas import tpu as pltpu

_B = 16384
_HIDDEN = 128
_NUM_CHARGES = 11
_BLOCK_B = 1024


def _fused_kernel(mz_ref, charge_ref, table_ref, w1_ref, b1_ref, w2_ref,
                  b2_ref, out_ref):
    mz = mz_ref[:]
    h = jnp.maximum(mz[:, None] * w1_ref[0][None, :] + b1_ref[:][None, :], 0.0)
    emb0 = jax.lax.dot_general(
        h, w2_ref[:],
        dimension_numbers=(((1,), (0,)), ((), ())),
        preferred_element_type=jnp.float32,
    ) + b2_ref[:][None, :]

    charge = charge_ref[:]
    classes = jax.lax.broadcasted_iota(jnp.int32, (charge.shape[0],
                                                   _NUM_CHARGES), 1)
    onehot = (charge[:, None] == classes).astype(jnp.float32)
    emb1 = jax.lax.dot_general(
        onehot, table_ref[:],
        dimension_numbers=(((1,), (0,)), ((), ())),
        preferred_element_type=jnp.float32,
    )

    out_ref[:, :_HIDDEN] = emb0
    out_ref[:, _HIDDEN:] = emb1


@jax.jit
def kernel(precursor_mz, charge, charge_table, W1, b1, W2, b2):
    charge = charge.astype(jnp.int32)
    grid = (_B // _BLOCK_B,)
    out = pl.pallas_call(
        _fused_kernel,
        grid=grid,
        in_specs=[
            pl.BlockSpec((_BLOCK_B,), lambda i: (i,)),
            pl.BlockSpec((_BLOCK_B,), lambda i: (i,)),
            pl.BlockSpec(charge_table.shape, lambda i: (0, 0)),
            pl.BlockSpec(W1.shape, lambda i: (0, 0)),
            pl.BlockSpec(b1.shape, lambda i: (0,)),
            pl.BlockSpec(W2.shape, lambda i: (0, 0)),
            pl.BlockSpec(b2.shape, lambda i: (0,)),
        ],
        out_specs=pl.BlockSpec((_BLOCK_B, 2 * _HIDDEN), lambda i: (i, 0)),
        out_shape=jax.ShapeDtypeStruct((_B, 2 * _HIDDEN), jnp.float32),
        compiler_params=pltpu.CompilerParams(
            dimension_semantics=("arbitrary",),
        ),
    )(precursor_mz, charge, charge_table, W1, b1, W2, b2)
    return out.reshape(_B, 2, _HIDDEN)


# speedup vs baseline: 3.5671x; 1.1943x over previous
import functools

import jax
import jax.numpy as jnp
from jax.experimental import pallas as pl
from jax.experimental.pallas import tpu as pltpu

_B = 16384
_HIDDEN = 128
_NUM_CHARGES = 11
_BLOCK_B = 4096


def _fused_kernel(mz_ref, charge_ref, table_ref, w1_ref, b1_ref, w2_ref,
                  b2_ref, out_ref):
    mz = mz_ref[:]
    h = jnp.maximum(mz[:, None] * w1_ref[0][None, :] + b1_ref[:][None, :], 0.0)
    emb0 = jax.lax.dot_general(
        h, w2_ref[:],
        dimension_numbers=(((1,), (0,)), ((), ())),
        preferred_element_type=jnp.float32,
    ) + b2_ref[:][None, :]

    charge = charge_ref[:]
    classes = jax.lax.broadcasted_iota(jnp.int32, (charge.shape[0],
                                                   _NUM_CHARGES), 1)
    onehot = (charge[:, None] == classes).astype(jnp.float32)
    emb1 = jax.lax.dot_general(
        onehot, table_ref[:],
        dimension_numbers=(((1,), (0,)), ((), ())),
        preferred_element_type=jnp.float32,
    )

    out_ref[:, :_HIDDEN] = emb0
    out_ref[:, _HIDDEN:] = emb1


@jax.jit
def kernel(precursor_mz, charge, charge_table, W1, b1, W2, b2):
    charge = charge.astype(jnp.int32)
    grid = (_B // _BLOCK_B,)
    out = pl.pallas_call(
        _fused_kernel,
        grid=grid,
        in_specs=[
            pl.BlockSpec((_BLOCK_B,), lambda i: (i,)),
            pl.BlockSpec((_BLOCK_B,), lambda i: (i,)),
            pl.BlockSpec(charge_table.shape, lambda i: (0, 0)),
            pl.BlockSpec(W1.shape, lambda i: (0, 0)),
            pl.BlockSpec(b1.shape, lambda i: (0,)),
            pl.BlockSpec(W2.shape, lambda i: (0, 0)),
            pl.BlockSpec(b2.shape, lambda i: (0,)),
        ],
        out_specs=pl.BlockSpec((_BLOCK_B, 2 * _HIDDEN), lambda i: (i, 0)),
        out_shape=jax.ShapeDtypeStruct((_B, 2 * _HIDDEN), jnp.float32),
        compiler_params=pltpu.CompilerParams(
            dimension_semantics=("arbitrary",),
        ),
    )(precursor_mz, charge, charge_table, W1, b1, W2, b2)
    return out.reshape(_B, 2, _HIDDEN)
